# baseline (speedup 1.0000x reference)
import functools

import jax
import jax.numpy as jnp
from jax import lax
from jax.experimental import pallas as pl
from jax.experimental.pallas import tpu as pltpu
from jax.experimental.pallas import tpu_sc as plsc

N = 10000
E = 320000
D = 128

NC = 2
NS = 16
NW = NC * NS
EPW = E // NW
C = 80
CPW = EPW // C
RPT = 624
TAIL = N - NS * RPT

_MESH = plsc.VectorSubcoreMesh(core_axis_name="c", subcore_axis_name="s",
                               num_cores=NC, num_subcores=NS)


def _init_shared(zeros_nw, acc_sh, s):
  r0 = s * RPT
  pltpu.sync_copy(zeros_nw.at[pl.ds(r0, RPT)], acc_sh.at[pl.ds(r0, RPT)])

  @pl.when(s == 0)
  def _():
    pltpu.sync_copy(zeros_nw.at[pl.ds(NS * RPT, TAIL)],
                    acc_sh.at[pl.ds(NS * RPT, TAIL)])


def _writeback(acc_sh, out_acc, c, s):
  r0 = s * RPT
  o0 = c * N + r0
  pltpu.sync_copy(acc_sh.at[pl.ds(r0, RPT)], out_acc.at[pl.ds(o0, RPT)])

  @pl.when(s == 0)
  def _():
    pltpu.sync_copy(acc_sh.at[pl.ds(NS * RPT, TAIL)],
                    out_acc.at[pl.ds(c * N + NS * RPT, TAIL)])


def _agg_body(y, src, dst, zeros_nw, out_acc, idx_s, idx_d, bufs, gsems,
              ssems, acc_sh):
  c = lax.axis_index("c")
  s = lax.axis_index("s")
  w = s * NC + c
  _init_shared(zeros_nw, acc_sh, s)
  plsc.subcore_barrier()

  e0 = w * EPW

  def stage(k, b):
    base = e0 + k * C
    pltpu.sync_copy(src.at[pl.ds(base, C)], idx_s[b])
    pltpu.sync_copy(dst.at[pl.ds(base, C)], idx_d[b])

  def gather(b):
    pltpu.async_copy(y.at[idx_s[b]], bufs[b], gsems[b])

  def scatter(b):
    pltpu.async_copy(bufs[b], acc_sh.at[idx_d[b]], ssems[b], add=True)

  def wait_g(b):
    pltpu.make_async_copy(y.at[idx_s[b]], bufs[b], gsems[b]).wait()

  def wait_s(b):
    pltpu.make_async_copy(bufs[b], acc_sh.at[idx_d[b]], ssems[b]).wait()

  for b in range(4):
    stage(b, b)
    gather(b)

  def step(i, carry):
    for b in range(4):
      wait_g(b)
      scatter(b)
    for b in range(4):
      wait_s(b)
      stage(4 * i + 4 + b, b)
      gather(b)
    return carry

  lax.fori_loop(0, (CPW - 5) // 4, step, 0)
  for b in range(4):
    wait_g(b)
    scatter(b)
  wait_s(0)
  stage(CPW - 1, 0)
  gather(0)
  wait_g(0)
  scatter(0)
  for b in range(4):
    wait_s(b)
  plsc.subcore_barrier()
  _writeback(acc_sh, out_acc, c, s)


_agg = pl.kernel(
    _agg_body,
    out_type=jax.ShapeDtypeStruct((NC * N, D), jnp.float32),
    mesh=_MESH,
    scratch_types=(
        [pltpu.VMEM((C,), jnp.int32)] * 4,
        [pltpu.VMEM((C,), jnp.int32)] * 4,
        [pltpu.VMEM((C, D), jnp.float32)] * 4,
        [pltpu.SemaphoreType.DMA] * 4,
        [pltpu.SemaphoreType.DMA] * 4,
        pltpu.VMEM_SHARED((N, D), jnp.float32),
    ))


def _cnt_body(dst, zeros_nw, ones_cd, out_cnt, idx_d, ones_v, ssems,
              cnt_sh):
  c = lax.axis_index("c")
  s = lax.axis_index("s")
  w = s * NC + c
  _init_shared(zeros_nw, cnt_sh, s)
  pltpu.sync_copy(ones_cd, ones_v)
  plsc.subcore_barrier()

  e0 = w * EPW

  def stage(k, b):
    pltpu.sync_copy(dst.at[pl.ds(e0 + k * C, C)], idx_d[b])

  def scatter(b):
    pltpu.async_copy(ones_v, cnt_sh.at[idx_d[b]], ssems[b], add=True)

  def wait_s(b):
    pltpu.make_async_copy(ones_v, cnt_sh.at[idx_d[b]], ssems[b]).wait()

  for b in range(4):
    stage(b, b)
    scatter(b)

  def step(i, carry):
    for b in range(4):
      wait_s(b)
      stage(4 * i + 4 + b, b)
      scatter(b)
    return carry

  lax.fori_loop(0, (CPW - 5) // 4, step, 0)
  wait_s(0)
  stage(CPW - 1, 0)
  scatter(0)
  for b in range(4):
    wait_s(b)
  plsc.subcore_barrier()
  _writeback(cnt_sh, out_cnt, c, s)


_cnt_agg = pl.kernel(
    _cnt_body,
    out_type=jax.ShapeDtypeStruct((NC * N, D), jnp.float32),
    mesh=_MESH,
    scratch_types=(
        [pltpu.VMEM((C,), jnp.int32)] * 4,
        pltpu.VMEM((C, D), jnp.float32),
        [pltpu.SemaphoreType.DMA] * 4,
        pltpu.VMEM_SHARED((N, D), jnp.float32),
    ))


def _dense1_body(a0_ref, a1_ref, c0_ref, c1_ref, x_ref, wl_ref, wr_ref,
                 bl_ref, g_ref, b_ref, o_ref, cnt_ref):
  acc = a0_ref[...] + a1_ref[...]
  cnt = jnp.maximum(c0_ref[:, 0:1] + c1_ref[:, 0:1], 1.0)
  mean = acc / cnt
  xb = x_ref[...]
  h = (jax.lax.dot(mean, wl_ref[...], precision=lax.Precision.HIGHEST,
                   preferred_element_type=jnp.float32)
       + bl_ref[...]
       + jax.lax.dot(xb, wr_ref[...], precision=lax.Precision.HIGHEST,
                     preferred_element_type=jnp.float32))
  mu = jnp.mean(h, axis=-1, keepdims=True)
  d = h - mu
  var = jnp.mean(d * d, axis=-1, keepdims=True)
  hn = d * jax.lax.rsqrt(var + 1e-5) * g_ref[...] + b_ref[...]
  o_ref[...] = jnp.maximum(hn, 0.0) + xb
  cnt_ref[...] = jnp.broadcast_to(cnt, cnt_ref.shape)


def _dense2_body(a0_ref, a1_ref, cnt_ref, x_ref, wl_ref, wr_ref, bl_ref,
                 g_ref, b_ref, o_ref):
  acc = a0_ref[...] + a1_ref[...]
  mean = acc / cnt_ref[:, 0:1]
  xb = x_ref[...]
  h = (jax.lax.dot(mean, wl_ref[...], precision=lax.Precision.HIGHEST,
                   preferred_element_type=jnp.float32)
       + bl_ref[...]
       + jax.lax.dot(xb, wr_ref[...], precision=lax.Precision.HIGHEST,
                     preferred_element_type=jnp.float32))
  mu = jnp.mean(h, axis=-1, keepdims=True)
  d = h - mu
  var = jnp.mean(d * d, axis=-1, keepdims=True)
  hn = d * jax.lax.rsqrt(var + 1e-5) * g_ref[...] + b_ref[...]
  o_ref[...] = hn + xb


R = 1000
NB = N // R

_W_SPEC = pl.BlockSpec((D, D), lambda i: (0, 0))
_B_SPEC = pl.BlockSpec((1, D), lambda i: (0, 0))
_ROW_SPEC = pl.BlockSpec((R, D), lambda i: (i, 0))
_LO_SPEC = pl.BlockSpec((R, D), lambda i: (i, 0))
_HI_SPEC = pl.BlockSpec((R, D), lambda i: (NB + i, 0))

_dense1 = pl.pallas_call(
    _dense1_body,
    grid=(NB,),
    in_specs=[
        _LO_SPEC, _HI_SPEC, _LO_SPEC, _HI_SPEC, _ROW_SPEC,
        _W_SPEC, _W_SPEC, _B_SPEC, _B_SPEC, _B_SPEC,
    ],
    out_specs=[
        _ROW_SPEC,
        pl.BlockSpec((R, 8), lambda i: (i, 0)),
    ],
    out_shape=[
        jax.ShapeDtypeStruct((N, D), jnp.float32),
        jax.ShapeDtypeStruct((N, 8), jnp.float32),
    ],
)

_dense2 = pl.pallas_call(
    _dense2_body,
    grid=(NB,),
    in_specs=[
        _LO_SPEC, _HI_SPEC,
        pl.BlockSpec((R, 8), lambda i: (i, 0)),
        _ROW_SPEC,
        _W_SPEC, _W_SPEC, _B_SPEC, _B_SPEC, _B_SPEC,
    ],
    out_specs=_ROW_SPEC,
    out_shape=jax.ShapeDtypeStruct((N, D), jnp.float32),
)


@jax.jit
def kernel(x, edge_index, Wl1, bl1, Wr1, g1, b1, Wl2, bl2, Wr2, g2, b2):
  src = edge_index[0]
  dst = edge_index[1]
  zeros_nd = jnp.zeros((N, D), jnp.float32)
  ones_cd = jnp.ones((C, D), jnp.float32)

  cntp = _cnt_agg(dst, zeros_nd, ones_cd)
  acc1 = _agg(x, src, dst, zeros_nd)
  h, cnt8 = _dense1(acc1, acc1, cntp, cntp, x, Wl1.T, Wr1.T,
                    bl1.reshape(1, D), g1.reshape(1, D), b1.reshape(1, D))
  acc2 = _agg(h, src, dst, zeros_nd)
  out = _dense2(acc2, acc2, cnt8, h, Wl2.T, Wr2.T, bl2.reshape(1, D),
                g2.reshape(1, D), b2.reshape(1, D))
  return out


# speedup vs baseline: 1.0857x; 1.0857x over previous
import functools

import jax
import jax.numpy as jnp
from jax import lax
from jax.experimental import pallas as pl
from jax.experimental.pallas import tpu as pltpu
from jax.experimental.pallas import tpu_sc as plsc

N = 10000
E = 320000
D = 128

NC = 2
NS = 16
NW = NC * NS
EPW = E // NW
C = 80
CPW = EPW // C
CL = 125
CPL = EPW // CL
PH = CPL // 2
RPT = 624
TAIL = N - NS * RPT

_MESH = plsc.VectorSubcoreMesh(core_axis_name="c", subcore_axis_name="s",
                               num_cores=NC, num_subcores=NS)


def _init_shared(zeros_nw, acc_sh, s):
  r0 = s * RPT
  pltpu.sync_copy(zeros_nw.at[pl.ds(r0, RPT)], acc_sh.at[pl.ds(r0, RPT)])

  @pl.when(s == 0)
  def _():
    pltpu.sync_copy(zeros_nw.at[pl.ds(NS * RPT, TAIL)],
                    acc_sh.at[pl.ds(NS * RPT, TAIL)])


def _writeback(acc_sh, out_acc, c, s):
  r0 = s * RPT
  o0 = c * N + r0
  pltpu.sync_copy(acc_sh.at[pl.ds(r0, RPT)], out_acc.at[pl.ds(o0, RPT)])

  @pl.when(s == 0)
  def _():
    pltpu.sync_copy(acc_sh.at[pl.ds(NS * RPT, TAIL)],
                    out_acc.at[pl.ds(c * N + NS * RPT, TAIL)])


def _agg_body(y, src4, dst3, zeros_nw, out_acc, idx_s, idx_d, buf0, buf1,
              g0, g1, s0, s1, acc_sh):
  c = lax.axis_index("c")
  s = lax.axis_index("s")
  w = s * NC + c
  _init_shared(zeros_nw, acc_sh, s)
  pltpu.sync_copy(dst3.at[w], idx_d)
  plsc.subcore_barrier()

  def gather(j, buf, sem):
    pltpu.async_copy(y.at[idx_s.at[j]], buf, sem)

  def scatter(g, buf, sem):
    pltpu.async_copy(buf, acc_sh.at[idx_d.at[g]], sem, add=True)

  def wait_g(buf, sem):
    pltpu.make_async_copy(y.at[idx_s.at[0]], buf, sem).wait()

  def wait_s(buf, sem):
    pltpu.make_async_copy(buf, acc_sh.at[idx_d.at[0]], sem).wait()

  for p in range(2):
    pltpu.sync_copy(src4.at[2 * w + p], idx_s)
    if p > 0:
      wait_s(buf0, s0)
    gather(0, buf0, g0)
    if p > 0:
      wait_s(buf1, s1)
    gather(1, buf1, g1)
    base = p * PH

    def step(i, carry):
      j0 = 2 * i
      wait_g(buf0, g0)
      scatter(base + j0, buf0, s0)
      wait_g(buf1, g1)
      scatter(base + j0 + 1, buf1, s1)
      wait_s(buf0, s0)
      gather(j0 + 2, buf0, g0)
      wait_s(buf1, s1)
      gather(j0 + 3, buf1, g1)
      return carry

    lax.fori_loop(0, (PH - 2) // 2, step, 0)
    wait_g(buf0, g0)
    scatter(base + PH - 2, buf0, s0)
    wait_g(buf1, g1)
    scatter(base + PH - 1, buf1, s1)

  wait_s(buf0, s0)
  wait_s(buf1, s1)
  plsc.subcore_barrier()
  _writeback(acc_sh, out_acc, c, s)


_agg = pl.kernel(
    _agg_body,
    out_type=jax.ShapeDtypeStruct((NC * N, D), jnp.float32),
    mesh=_MESH,
    scratch_types=(
        pltpu.VMEM((PH, CL), jnp.int32),
        pltpu.VMEM((CPL, CL), jnp.int32),
        pltpu.VMEM((CL, D), jnp.float32),
        pltpu.VMEM((CL, D), jnp.float32),
        pltpu.SemaphoreType.DMA,
        pltpu.SemaphoreType.DMA,
        pltpu.SemaphoreType.DMA,
        pltpu.SemaphoreType.DMA,
        pltpu.VMEM_SHARED((N, D), jnp.float32),
    ))


def _cnt_body(dst3, zeros_nw, ones_cd, out_cnt, idx_d, ones_v, ssems,
              cnt_sh):
  c = lax.axis_index("c")
  s = lax.axis_index("s")
  w = s * NC + c
  _init_shared(zeros_nw, cnt_sh, s)
  pltpu.sync_copy(ones_cd, ones_v)
  pltpu.sync_copy(dst3.at[w], idx_d)
  plsc.subcore_barrier()

  def scatter(g, b):
    pltpu.async_copy(ones_v, cnt_sh.at[idx_d.at[g]], ssems[b], add=True)

  def wait_s(b):
    pltpu.make_async_copy(ones_v, cnt_sh.at[idx_d.at[0]], ssems[b]).wait()

  for b in range(4):
    scatter(b, b)

  def step(i, carry):
    for b in range(4):
      wait_s(b)
      scatter(4 * i + 4 + b, b)
    return carry

  lax.fori_loop(0, (CPL - 4) // 4, step, 0)
  for b in range(4):
    wait_s(b)
  plsc.subcore_barrier()
  _writeback(cnt_sh, out_cnt, c, s)


_cnt_agg = pl.kernel(
    _cnt_body,
    out_type=jax.ShapeDtypeStruct((NC * N, D), jnp.float32),
    mesh=_MESH,
    scratch_types=(
        pltpu.VMEM((CPL, CL), jnp.int32),
        pltpu.VMEM((CL, D), jnp.float32),
        [pltpu.SemaphoreType.DMA] * 4,
        pltpu.VMEM_SHARED((N, D), jnp.float32),
    ))


def _dense1_body(a0_ref, a1_ref, c0_ref, c1_ref, x_ref, wl_ref, wr_ref,
                 bl_ref, g_ref, b_ref, o_ref, cnt_ref):
  acc = a0_ref[...] + a1_ref[...]
  cnt = jnp.maximum(c0_ref[:, 0:1] + c1_ref[:, 0:1], 1.0)
  mean = acc / cnt
  xb = x_ref[...]
  h = (jax.lax.dot(mean, wl_ref[...], precision=lax.Precision.HIGHEST,
                   preferred_element_type=jnp.float32)
       + bl_ref[...]
       + jax.lax.dot(xb, wr_ref[...], precision=lax.Precision.HIGHEST,
                     preferred_element_type=jnp.float32))
  mu = jnp.mean(h, axis=-1, keepdims=True)
  d = h - mu
  var = jnp.mean(d * d, axis=-1, keepdims=True)
  hn = d * jax.lax.rsqrt(var + 1e-5) * g_ref[...] + b_ref[...]
  o_ref[...] = jnp.maximum(hn, 0.0) + xb
  cnt_ref[...] = jnp.broadcast_to(cnt, cnt_ref.shape)


def _dense2_body(a0_ref, a1_ref, cnt_ref, x_ref, wl_ref, wr_ref, bl_ref,
                 g_ref, b_ref, o_ref):
  acc = a0_ref[...] + a1_ref[...]
  mean = acc / cnt_ref[:, 0:1]
  xb = x_ref[...]
  h = (jax.lax.dot(mean, wl_ref[...], precision=lax.Precision.HIGHEST,
                   preferred_element_type=jnp.float32)
       + bl_ref[...]
       + jax.lax.dot(xb, wr_ref[...], precision=lax.Precision.HIGHEST,
                     preferred_element_type=jnp.float32))
  mu = jnp.mean(h, axis=-1, keepdims=True)
  d = h - mu
  var = jnp.mean(d * d, axis=-1, keepdims=True)
  hn = d * jax.lax.rsqrt(var + 1e-5) * g_ref[...] + b_ref[...]
  o_ref[...] = hn + xb


R = 1000
NB = N // R

_W_SPEC = pl.BlockSpec((D, D), lambda i: (0, 0))
_B_SPEC = pl.BlockSpec((1, D), lambda i: (0, 0))
_ROW_SPEC = pl.BlockSpec((R, D), lambda i: (i, 0))
_LO_SPEC = pl.BlockSpec((R, D), lambda i: (i, 0))
_HI_SPEC = pl.BlockSpec((R, D), lambda i: (NB + i, 0))

_dense1 = pl.pallas_call(
    _dense1_body,
    grid=(NB,),
    in_specs=[
        _LO_SPEC, _HI_SPEC, _LO_SPEC, _HI_SPEC, _ROW_SPEC,
        _W_SPEC, _W_SPEC, _B_SPEC, _B_SPEC, _B_SPEC,
    ],
    out_specs=[
        _ROW_SPEC,
        pl.BlockSpec((R, 8), lambda i: (i, 0)),
    ],
    out_shape=[
        jax.ShapeDtypeStruct((N, D), jnp.float32),
        jax.ShapeDtypeStruct((N, 8), jnp.float32),
    ],
)

_dense2 = pl.pallas_call(
    _dense2_body,
    grid=(NB,),
    in_specs=[
        _LO_SPEC, _HI_SPEC,
        pl.BlockSpec((R, 8), lambda i: (i, 0)),
        _ROW_SPEC,
        _W_SPEC, _W_SPEC, _B_SPEC, _B_SPEC, _B_SPEC,
    ],
    out_specs=_ROW_SPEC,
    out_shape=jax.ShapeDtypeStruct((N, D), jnp.float32),
)


@jax.jit
def kernel(x, edge_index, Wl1, bl1, Wr1, g1, b1, Wl2, bl2, Wr2, g2, b2):
  src = edge_index[0]
  dst = edge_index[1]
  src4 = src.reshape(NW * 2, PH, CL)
  dst3 = dst.reshape(NW, CPL, CL)
  zeros_nd = jnp.zeros((N, D), jnp.float32)
  ones_cd = jnp.ones((CL, D), jnp.float32)

  cntp = _cnt_agg(dst3, zeros_nd, ones_cd)
  acc1 = _agg(x, src4, dst3, zeros_nd)
  h, cnt8 = _dense1(acc1, acc1, cntp, cntp, x, Wl1.T, Wr1.T,
                    bl1.reshape(1, D), g1.reshape(1, D), b1.reshape(1, D))
  acc2 = _agg(h, src4, dst3, zeros_nd)
  out = _dense2(acc2, acc2, cnt8, h, Wl2.T, Wr2.T, bl2.reshape(1, D),
                g2.reshape(1, D), b2.reshape(1, D))
  return out


# speedup vs baseline: 1.0911x; 1.0049x over previous
import functools

import jax
import jax.numpy as jnp
from jax import lax
from jax.experimental import pallas as pl
from jax.experimental.pallas import tpu as pltpu
from jax.experimental.pallas import tpu_sc as plsc

N = 10000
E = 320000
D = 128

NC = 2
NS = 16
NW = NC * NS
EPW = E // NW
C = 80
CPW = EPW // C
CL = 125
CPL = EPW // CL
PH = CPL // 2
RPT = 624
TAIL = N - NS * RPT

_MESH = plsc.VectorSubcoreMesh(core_axis_name="c", subcore_axis_name="s",
                               num_cores=NC, num_subcores=NS)


def _init_shared(zeros_nw, acc_sh, s):
  r0 = s * RPT
  pltpu.sync_copy(zeros_nw.at[pl.ds(r0, RPT)], acc_sh.at[pl.ds(r0, RPT)])

  @pl.when(s == 0)
  def _():
    pltpu.sync_copy(zeros_nw.at[pl.ds(NS * RPT, TAIL)],
                    acc_sh.at[pl.ds(NS * RPT, TAIL)])


def _writeback(acc_sh, out_acc, c, s):
  r0 = s * RPT
  o0 = c * N + r0
  pltpu.sync_copy(acc_sh.at[pl.ds(r0, RPT)], out_acc.at[pl.ds(o0, RPT)])

  @pl.when(s == 0)
  def _():
    pltpu.sync_copy(acc_sh.at[pl.ds(NS * RPT, TAIL)],
                    out_acc.at[pl.ds(c * N + NS * RPT, TAIL)])


def _agg_body(y, src4, dst3, zeros_nw, out_acc, idx_s, idx_d, buf0, buf1,
              g0, g1, s0, s1, acc_sh):
  c = lax.axis_index("c")
  s = lax.axis_index("s")
  w = s * NC + c
  _init_shared(zeros_nw, acc_sh, s)
  pltpu.sync_copy(dst3.at[w], idx_d)
  plsc.subcore_barrier()

  def gather(j, buf, sem):
    pltpu.async_copy(y.at[idx_s.at[j]], buf, sem)

  def scatter(g, buf, sem):
    pltpu.async_copy(buf, acc_sh.at[idx_d.at[g]], sem, add=True)

  def wait_g(buf, sem):
    pltpu.make_async_copy(y.at[idx_s.at[0]], buf, sem).wait()

  def wait_s(buf, sem):
    pltpu.make_async_copy(buf, acc_sh.at[idx_d.at[0]], sem).wait()

  for p in range(2):
    pltpu.sync_copy(src4.at[2 * w + p], idx_s)
    if p > 0:
      wait_s(buf0, s0)
    gather(0, buf0, g0)
    if p > 0:
      wait_s(buf1, s1)
    gather(1, buf1, g1)
    base = p * PH

    def step(i, carry):
      j0 = 2 * i
      wait_g(buf0, g0)
      scatter(base + j0, buf0, s0)
      wait_g(buf1, g1)
      scatter(base + j0 + 1, buf1, s1)
      wait_s(buf0, s0)
      gather(j0 + 2, buf0, g0)
      wait_s(buf1, s1)
      gather(j0 + 3, buf1, g1)
      return carry

    lax.fori_loop(0, (PH - 2) // 2, step, 0)
    wait_g(buf0, g0)
    scatter(base + PH - 2, buf0, s0)
    wait_g(buf1, g1)
    scatter(base + PH - 1, buf1, s1)

  wait_s(buf0, s0)
  wait_s(buf1, s1)
  plsc.subcore_barrier()
  _writeback(acc_sh, out_acc, c, s)


_agg = pl.kernel(
    _agg_body,
    out_type=jax.ShapeDtypeStruct((NC * N, D), jnp.float32),
    mesh=_MESH,
    scratch_types=(
        pltpu.VMEM((PH, CL), jnp.int32),
        pltpu.VMEM((CPL, CL), jnp.int32),
        pltpu.VMEM((CL, D), jnp.float32),
        pltpu.VMEM((CL, D), jnp.float32),
        pltpu.SemaphoreType.DMA,
        pltpu.SemaphoreType.DMA,
        pltpu.SemaphoreType.DMA,
        pltpu.SemaphoreType.DMA,
        pltpu.VMEM_SHARED((N, D), jnp.float32),
    ))


def _agg_cnt_body(y, src4, dst3, zeros_nw, ones_cd, out_acc, out_cnt,
                  idx_s, idx_d, buf0, buf1, g0, g1, s0, s1, acc_sh):
  c = lax.axis_index("c")
  s = lax.axis_index("s")
  w = s * NC + c
  _init_shared(zeros_nw, acc_sh, s)
  pltpu.sync_copy(dst3.at[w], idx_d)
  plsc.subcore_barrier()

  def gather(j, buf, sem):
    pltpu.async_copy(y.at[idx_s.at[j]], buf, sem)

  def scatter(g, buf, sem):
    pltpu.async_copy(buf, acc_sh.at[idx_d.at[g]], sem, add=True)

  def wait_g(buf, sem):
    pltpu.make_async_copy(y.at[idx_s.at[0]], buf, sem).wait()

  def wait_s(buf, sem):
    pltpu.make_async_copy(buf, acc_sh.at[idx_d.at[0]], sem).wait()

  for p in range(2):
    pltpu.sync_copy(src4.at[2 * w + p], idx_s)
    if p > 0:
      wait_s(buf0, s0)
    gather(0, buf0, g0)
    if p > 0:
      wait_s(buf1, s1)
    gather(1, buf1, g1)
    base = p * PH

    def step(i, carry):
      j0 = 2 * i
      wait_g(buf0, g0)
      scatter(base + j0, buf0, s0)
      wait_g(buf1, g1)
      scatter(base + j0 + 1, buf1, s1)
      wait_s(buf0, s0)
      gather(j0 + 2, buf0, g0)
      wait_s(buf1, s1)
      gather(j0 + 3, buf1, g1)
      return carry

    lax.fori_loop(0, (PH - 2) // 2, step, 0)
    wait_g(buf0, g0)
    scatter(base + PH - 2, buf0, s0)
    wait_g(buf1, g1)
    scatter(base + PH - 1, buf1, s1)

  wait_s(buf0, s0)
  wait_s(buf1, s1)
  plsc.subcore_barrier()
  _writeback(acc_sh, out_acc, c, s)
  plsc.subcore_barrier()

  _init_shared(zeros_nw, acc_sh, s)
  pltpu.sync_copy(ones_cd, buf0)
  plsc.subcore_barrier()

  def cscatter(g, sem):
    pltpu.async_copy(buf0, acc_sh.at[idx_d.at[g]], sem, add=True)

  def cwait(sem):
    pltpu.make_async_copy(buf0, acc_sh.at[idx_d.at[0]], sem).wait()

  csems = (g0, g1, s0, s1)
  for b in range(4):
    cscatter(b, csems[b])

  def cstep(i, carry):
    for b in range(4):
      cwait(csems[b])
      cscatter(4 * i + 4 + b, csems[b])
    return carry

  lax.fori_loop(0, (CPL - 4) // 4, cstep, 0)
  for b in range(4):
    cwait(csems[b])
  plsc.subcore_barrier()
  _writeback(acc_sh, out_cnt, c, s)


_agg_cnt = pl.kernel(
    _agg_cnt_body,
    out_type=(jax.ShapeDtypeStruct((NC * N, D), jnp.float32),
              jax.ShapeDtypeStruct((NC * N, D), jnp.float32)),
    mesh=_MESH,
    scratch_types=(
        pltpu.VMEM((PH, CL), jnp.int32),
        pltpu.VMEM((CPL, CL), jnp.int32),
        pltpu.VMEM((CL, D), jnp.float32),
        pltpu.VMEM((CL, D), jnp.float32),
        pltpu.SemaphoreType.DMA,
        pltpu.SemaphoreType.DMA,
        pltpu.SemaphoreType.DMA,
        pltpu.SemaphoreType.DMA,
        pltpu.VMEM_SHARED((N, D), jnp.float32),
    ))


def _dense1_body(a0_ref, a1_ref, c0_ref, c1_ref, x_ref, wl_ref, wr_ref,
                 bl_ref, g_ref, b_ref, o_ref, cnt_ref):
  acc = a0_ref[...] + a1_ref[...]
  cnt = jnp.maximum(c0_ref[:, 0:1] + c1_ref[:, 0:1], 1.0)
  mean = acc / cnt
  xb = x_ref[...]
  h = (jax.lax.dot(mean, wl_ref[...], precision=lax.Precision.HIGHEST,
                   preferred_element_type=jnp.float32)
       + bl_ref[...]
       + jax.lax.dot(xb, wr_ref[...], precision=lax.Precision.HIGHEST,
                     preferred_element_type=jnp.float32))
  mu = jnp.mean(h, axis=-1, keepdims=True)
  d = h - mu
  var = jnp.mean(d * d, axis=-1, keepdims=True)
  hn = d * jax.lax.rsqrt(var + 1e-5) * g_ref[...] + b_ref[...]
  o_ref[...] = jnp.maximum(hn, 0.0) + xb
  cnt_ref[...] = jnp.broadcast_to(cnt, cnt_ref.shape)


def _dense2_body(a0_ref, a1_ref, cnt_ref, x_ref, wl_ref, wr_ref, bl_ref,
                 g_ref, b_ref, o_ref):
  acc = a0_ref[...] + a1_ref[...]
  mean = acc / cnt_ref[:, 0:1]
  xb = x_ref[...]
  h = (jax.lax.dot(mean, wl_ref[...], precision=lax.Precision.HIGHEST,
                   preferred_element_type=jnp.float32)
       + bl_ref[...]
       + jax.lax.dot(xb, wr_ref[...], precision=lax.Precision.HIGHEST,
                     preferred_element_type=jnp.float32))
  mu = jnp.mean(h, axis=-1, keepdims=True)
  d = h - mu
  var = jnp.mean(d * d, axis=-1, keepdims=True)
  hn = d * jax.lax.rsqrt(var + 1e-5) * g_ref[...] + b_ref[...]
  o_ref[...] = hn + xb


R = 1000
NB = N // R

_W_SPEC = pl.BlockSpec((D, D), lambda i: (0, 0))
_B_SPEC = pl.BlockSpec((1, D), lambda i: (0, 0))
_ROW_SPEC = pl.BlockSpec((R, D), lambda i: (i, 0))
_LO_SPEC = pl.BlockSpec((R, D), lambda i: (i, 0))
_HI_SPEC = pl.BlockSpec((R, D), lambda i: (NB + i, 0))

_dense1 = pl.pallas_call(
    _dense1_body,
    grid=(NB,),
    in_specs=[
        _LO_SPEC, _HI_SPEC, _LO_SPEC, _HI_SPEC, _ROW_SPEC,
        _W_SPEC, _W_SPEC, _B_SPEC, _B_SPEC, _B_SPEC,
    ],
    out_specs=[
        _ROW_SPEC,
        pl.BlockSpec((R, 8), lambda i: (i, 0)),
    ],
    out_shape=[
        jax.ShapeDtypeStruct((N, D), jnp.float32),
        jax.ShapeDtypeStruct((N, 8), jnp.float32),
    ],
)

_dense2 = pl.pallas_call(
    _dense2_body,
    grid=(NB,),
    in_specs=[
        _LO_SPEC, _HI_SPEC,
        pl.BlockSpec((R, 8), lambda i: (i, 0)),
        _ROW_SPEC,
        _W_SPEC, _W_SPEC, _B_SPEC, _B_SPEC, _B_SPEC,
    ],
    out_specs=_ROW_SPEC,
    out_shape=jax.ShapeDtypeStruct((N, D), jnp.float32),
)


@jax.jit
def kernel(x, edge_index, Wl1, bl1, Wr1, g1, b1, Wl2, bl2, Wr2, g2, b2):
  src = edge_index[0]
  dst = edge_index[1]
  src4 = src.reshape(NW * 2, PH, CL)
  dst3 = dst.reshape(NW, CPL, CL)
  zeros_nd = jnp.zeros((N, D), jnp.float32)
  ones_cd = jnp.ones((CL, D), jnp.float32)

  acc1, cntp = _agg_cnt(x, src4, dst3, zeros_nd, ones_cd)
  h, cnt8 = _dense1(acc1, acc1, cntp, cntp, x, Wl1.T, Wr1.T,
                    bl1.reshape(1, D), g1.reshape(1, D), b1.reshape(1, D))
  acc2 = _agg(h, src4, dst3, zeros_nd)
  out = _dense2(acc2, acc2, cnt8, h, Wl2.T, Wr2.T, bl2.reshape(1, D),
                g2.reshape(1, D), b2.reshape(1, D))
  return out


# speedup vs baseline: 1.1219x; 1.0283x over previous
import functools

import jax
import jax.numpy as jnp
from jax import lax
from jax.experimental import pallas as pl
from jax.experimental.pallas import tpu as pltpu
from jax.experimental.pallas import tpu_sc as plsc

N = 10000
E = 320000
D = 128

NC = 2
NS = 16
NW = NC * NS
EPW = E // NW
C = 80
CPW = EPW // C
CL = 125
CPL = EPW // CL
PH = CPL // 2
RPT = 624
TAIL = N - NS * RPT

_MESH = plsc.VectorSubcoreMesh(core_axis_name="c", subcore_axis_name="s",
                               num_cores=NC, num_subcores=NS)


def _init_shared(zeros_nw, acc_sh, s):
  r0 = s * RPT
  pltpu.sync_copy(zeros_nw.at[pl.ds(r0, RPT)], acc_sh.at[pl.ds(r0, RPT)])

  @pl.when(s == 0)
  def _():
    pltpu.sync_copy(zeros_nw.at[pl.ds(NS * RPT, TAIL)],
                    acc_sh.at[pl.ds(NS * RPT, TAIL)])


def _writeback(acc_sh, out_acc, c, s):
  r0 = s * RPT
  o0 = c * N + r0
  pltpu.sync_copy(acc_sh.at[pl.ds(r0, RPT)], out_acc.at[pl.ds(o0, RPT)])

  @pl.when(s == 0)
  def _():
    pltpu.sync_copy(acc_sh.at[pl.ds(NS * RPT, TAIL)],
                    out_acc.at[pl.ds(c * N + NS * RPT, TAIL)])


def _agg_body(y, src4, dst3, zeros_nw, out_acc, idx_s, idx_d, buf0, buf1,
              g0, g1, s0, s1, acc_sh):
  c = lax.axis_index("c")
  s = lax.axis_index("s")
  w = s * NC + c
  _init_shared(zeros_nw, acc_sh, s)
  pltpu.sync_copy(dst3.at[w], idx_d)
  plsc.subcore_barrier()

  def gather(j, buf, sem):
    pltpu.async_copy(y.at[idx_s.at[j]], buf, sem)

  def scatter(g, buf, sem):
    pltpu.async_copy(buf, acc_sh.at[idx_d.at[g]], sem, add=True)

  def wait_g(buf, sem):
    pltpu.make_async_copy(y.at[idx_s.at[0]], buf, sem).wait()

  def wait_s(buf, sem):
    pltpu.make_async_copy(buf, acc_sh.at[idx_d.at[0]], sem).wait()

  for p in range(2):
    pltpu.sync_copy(src4.at[2 * w + p], idx_s)
    if p > 0:
      wait_s(buf0, s0)
    gather(0, buf0, g0)
    if p > 0:
      wait_s(buf1, s1)
    gather(1, buf1, g1)
    base = p * PH

    def step(i, carry):
      j0 = 2 * i
      wait_g(buf0, g0)
      scatter(base + j0, buf0, s0)
      wait_g(buf1, g1)
      scatter(base + j0 + 1, buf1, s1)
      wait_s(buf0, s0)
      gather(j0 + 2, buf0, g0)
      wait_s(buf1, s1)
      gather(j0 + 3, buf1, g1)
      return carry

    lax.fori_loop(0, (PH - 2) // 2, step, 0)
    wait_g(buf0, g0)
    scatter(base + PH - 2, buf0, s0)
    wait_g(buf1, g1)
    scatter(base + PH - 1, buf1, s1)

  wait_s(buf0, s0)
  wait_s(buf1, s1)
  plsc.subcore_barrier()
  _writeback(acc_sh, out_acc, c, s)


_agg = pl.kernel(
    _agg_body,
    out_type=jax.ShapeDtypeStruct((NC * N, D), jnp.float32),
    mesh=_MESH,
    scratch_types=(
        pltpu.VMEM((PH, CL), jnp.int32),
        pltpu.VMEM((CPL, CL), jnp.int32),
        pltpu.VMEM((CL, D), jnp.float32),
        pltpu.VMEM((CL, D), jnp.float32),
        pltpu.SemaphoreType.DMA,
        pltpu.SemaphoreType.DMA,
        pltpu.SemaphoreType.DMA,
        pltpu.SemaphoreType.DMA,
        pltpu.VMEM_SHARED((N, D), jnp.float32),
    ))


def _agg_cnt_body(y, src4, dst3, zeros_nw, ones_cd, out_acc, out_cnt,
                  idx_s, idx_d, buf0, buf1, g0, g1, s0, s1, acc_sh):
  c = lax.axis_index("c")
  s = lax.axis_index("s")
  w = s * NC + c
  _init_shared(zeros_nw, acc_sh, s)
  pltpu.sync_copy(dst3.at[w], idx_d)
  plsc.subcore_barrier()

  def gather(j, buf, sem):
    pltpu.async_copy(y.at[idx_s.at[j]], buf, sem)

  def scatter(g, buf, sem):
    pltpu.async_copy(buf, acc_sh.at[idx_d.at[g]], sem, add=True)

  def wait_g(buf, sem):
    pltpu.make_async_copy(y.at[idx_s.at[0]], buf, sem).wait()

  def wait_s(buf, sem):
    pltpu.make_async_copy(buf, acc_sh.at[idx_d.at[0]], sem).wait()

  for p in range(2):
    pltpu.sync_copy(src4.at[2 * w + p], idx_s)
    if p > 0:
      wait_s(buf0, s0)
    gather(0, buf0, g0)
    if p > 0:
      wait_s(buf1, s1)
    gather(1, buf1, g1)
    base = p * PH

    def step(i, carry):
      j0 = 2 * i
      wait_g(buf0, g0)
      scatter(base + j0, buf0, s0)
      wait_g(buf1, g1)
      scatter(base + j0 + 1, buf1, s1)
      wait_s(buf0, s0)
      gather(j0 + 2, buf0, g0)
      wait_s(buf1, s1)
      gather(j0 + 3, buf1, g1)
      return carry

    lax.fori_loop(0, (PH - 2) // 2, step, 0)
    wait_g(buf0, g0)
    scatter(base + PH - 2, buf0, s0)
    wait_g(buf1, g1)
    scatter(base + PH - 1, buf1, s1)

  wait_s(buf0, s0)
  wait_s(buf1, s1)
  plsc.subcore_barrier()
  _writeback(acc_sh, out_acc, c, s)
  plsc.subcore_barrier()

  _init_shared(zeros_nw, acc_sh, s)
  pltpu.sync_copy(ones_cd, buf0)
  plsc.subcore_barrier()

  def cscatter(g, sem):
    pltpu.async_copy(buf0, acc_sh.at[idx_d.at[g]], sem, add=True)

  def cwait(sem):
    pltpu.make_async_copy(buf0, acc_sh.at[idx_d.at[0]], sem).wait()

  csems = (g0, g1, s0, s1)
  for b in range(4):
    cscatter(b, csems[b])

  def cstep(i, carry):
    for b in range(4):
      cwait(csems[b])
      cscatter(4 * i + 4 + b, csems[b])
    return carry

  lax.fori_loop(0, (CPL - 4) // 4, cstep, 0)
  for b in range(4):
    cwait(csems[b])
  plsc.subcore_barrier()
  _writeback(acc_sh, out_cnt, c, s)


_agg_cnt = pl.kernel(
    _agg_cnt_body,
    out_type=(jax.ShapeDtypeStruct((NC * N, D), jnp.float32),
              jax.ShapeDtypeStruct((NC * N, D), jnp.float32)),
    mesh=_MESH,
    scratch_types=(
        pltpu.VMEM((PH, CL), jnp.int32),
        pltpu.VMEM((CPL, CL), jnp.int32),
        pltpu.VMEM((CL, D), jnp.float32),
        pltpu.VMEM((CL, D), jnp.float32),
        pltpu.SemaphoreType.DMA,
        pltpu.SemaphoreType.DMA,
        pltpu.SemaphoreType.DMA,
        pltpu.SemaphoreType.DMA,
        pltpu.VMEM_SHARED((N, D), jnp.float32),
    ))


def _dense1_body(a0_ref, a1_ref, c0_ref, c1_ref, x_ref, wl_ref, wr_ref,
                 bl_ref, g_ref, b_ref, o_ref, cnt_ref):
  acc = a0_ref[...] + a1_ref[...]
  cnt = jnp.maximum(c0_ref[:, 0:1] + c1_ref[:, 0:1], 1.0)
  mean = acc / cnt
  xb = x_ref[...]
  h = (jax.lax.dot(mean, wl_ref[...], precision=lax.Precision.HIGHEST,
                   preferred_element_type=jnp.float32)
       + bl_ref[...]
       + jax.lax.dot(xb, wr_ref[...], precision=lax.Precision.HIGHEST,
                     preferred_element_type=jnp.float32))
  mu = jnp.mean(h, axis=-1, keepdims=True)
  d = h - mu
  var = jnp.mean(d * d, axis=-1, keepdims=True)
  hn = d * jax.lax.rsqrt(var + 1e-5) * g_ref[...] + b_ref[...]
  o_ref[...] = jnp.maximum(hn, 0.0) + xb
  cnt_ref[...] = jnp.broadcast_to(cnt, cnt_ref.shape)


def _dense2_body(a0_ref, a1_ref, cnt_ref, x_ref, wl_ref, wr_ref, bl_ref,
                 g_ref, b_ref, o_ref):
  acc = a0_ref[...] + a1_ref[...]
  mean = acc / cnt_ref[:, 0:1]
  xb = x_ref[...]
  h = (jax.lax.dot(mean, wl_ref[...], precision=lax.Precision.HIGHEST,
                   preferred_element_type=jnp.float32)
       + bl_ref[...]
       + jax.lax.dot(xb, wr_ref[...], precision=lax.Precision.HIGHEST,
                     preferred_element_type=jnp.float32))
  mu = jnp.mean(h, axis=-1, keepdims=True)
  d = h - mu
  var = jnp.mean(d * d, axis=-1, keepdims=True)
  hn = d * jax.lax.rsqrt(var + 1e-5) * g_ref[...] + b_ref[...]
  o_ref[...] = hn + xb


R = 2000
NB = N // R

_W_SPEC = pl.BlockSpec((D, D), lambda i: (0, 0))
_B_SPEC = pl.BlockSpec((1, D), lambda i: (0, 0))
_ROW_SPEC = pl.BlockSpec((R, D), lambda i: (i, 0))
_LO_SPEC = pl.BlockSpec((R, D), lambda i: (i, 0))
_HI_SPEC = pl.BlockSpec((R, D), lambda i: (NB + i, 0))

_dense1 = pl.pallas_call(
    _dense1_body,
    grid=(NB,),
    in_specs=[
        _LO_SPEC, _HI_SPEC, _LO_SPEC, _HI_SPEC, _ROW_SPEC,
        _W_SPEC, _W_SPEC, _B_SPEC, _B_SPEC, _B_SPEC,
    ],
    out_specs=[
        _ROW_SPEC,
        pl.BlockSpec((R, 8), lambda i: (i, 0)),
    ],
    out_shape=[
        jax.ShapeDtypeStruct((N, D), jnp.float32),
        jax.ShapeDtypeStruct((N, 8), jnp.float32),
    ],
)

_dense2 = pl.pallas_call(
    _dense2_body,
    grid=(NB,),
    in_specs=[
        _LO_SPEC, _HI_SPEC,
        pl.BlockSpec((R, 8), lambda i: (i, 0)),
        _ROW_SPEC,
        _W_SPEC, _W_SPEC, _B_SPEC, _B_SPEC, _B_SPEC,
    ],
    out_specs=_ROW_SPEC,
    out_shape=jax.ShapeDtypeStruct((N, D), jnp.float32),
)


@jax.jit
def kernel(x, edge_index, Wl1, bl1, Wr1, g1, b1, Wl2, bl2, Wr2, g2, b2):
  src = edge_index[0]
  dst = edge_index[1]
  src4 = src.reshape(NW * 2, PH, CL)
  dst3 = dst.reshape(NW, CPL, CL)
  zeros_nd = jnp.zeros((N, D), jnp.float32)
  ones_cd = jnp.ones((CL, D), jnp.float32)

  acc1, cntp = _agg_cnt(x, src4, dst3, zeros_nd, ones_cd)
  h, cnt8 = _dense1(acc1, acc1, cntp, cntp, x, Wl1.T, Wr1.T,
                    bl1.reshape(1, D), g1.reshape(1, D), b1.reshape(1, D))
  acc2 = _agg(h, src4, dst3, zeros_nd)
  out = _dense2(acc2, acc2, cnt8, h, Wl2.T, Wr2.T, bl2.reshape(1, D),
                g2.reshape(1, D), b2.reshape(1, D))
  return out


# speedup vs baseline: 1.1223x; 1.0003x over previous
import jax
import jax.numpy as jnp
from jax import lax
from jax.experimental import pallas as pl
from jax.experimental.pallas import tpu as pltpu
from jax.experimental.pallas import tpu_sc as plsc

N = 10000
E = 320000
D = 128

NC = 2
NS = 16
NW = NC * NS
EPW = E // NW
CL = 125
CPL = EPW // CL
PH = CPL // 2
RPT = 624
TAIL = N - NS * RPT

_MESH = plsc.VectorSubcoreMesh(core_axis_name="c", subcore_axis_name="s",
                               num_cores=NC, num_subcores=NS)


def _init_shared(zeros_nw, acc_sh, s):
  r0 = s * RPT
  pltpu.sync_copy(zeros_nw.at[pl.ds(r0, RPT)], acc_sh.at[pl.ds(r0, RPT)])

  @pl.when(s == 0)
  def _():
    pltpu.sync_copy(zeros_nw.at[pl.ds(NS * RPT, TAIL)],
                    acc_sh.at[pl.ds(NS * RPT, TAIL)])


def _writeback(acc_sh, out_acc, c, s):
  r0 = s * RPT
  o0 = c * N + r0
  pltpu.sync_copy(acc_sh.at[pl.ds(r0, RPT)], out_acc.at[pl.ds(o0, RPT)])

  @pl.when(s == 0)
  def _():
    pltpu.sync_copy(acc_sh.at[pl.ds(NS * RPT, TAIL)],
                    out_acc.at[pl.ds(c * N + NS * RPT, TAIL)])


def _agg_body(y, src4, dst3, zeros_nw, out_acc, idx_s, idx_d, buf0, buf1,
              g0, g1, s0, s1, acc_sh):
  c = lax.axis_index("c")
  s = lax.axis_index("s")
  w = s * NC + c
  _init_shared(zeros_nw, acc_sh, s)
  pltpu.sync_copy(dst3.at[w], idx_d)
  plsc.subcore_barrier()

  def gather(j, buf, sem):
    pltpu.async_copy(y.at[idx_s.at[j]], buf, sem)

  def scatter(g, buf, sem):
    pltpu.async_copy(buf, acc_sh.at[idx_d.at[g]], sem, add=True)

  def wait_g(buf, sem):
    pltpu.make_async_copy(y.at[idx_s.at[0]], buf, sem).wait()

  def wait_s(buf, sem):
    pltpu.make_async_copy(buf, acc_sh.at[idx_d.at[0]], sem).wait()

  for p in range(2):
    pltpu.sync_copy(src4.at[2 * w + p], idx_s)
    if p > 0:
      wait_s(buf0, s0)
    gather(0, buf0, g0)
    if p > 0:
      wait_s(buf1, s1)
    gather(1, buf1, g1)
    base = p * PH

    def step(i, carry):
      j0 = 2 * i
      wait_g(buf0, g0)
      scatter(base + j0, buf0, s0)
      wait_g(buf1, g1)
      scatter(base + j0 + 1, buf1, s1)
      wait_s(buf0, s0)
      gather(j0 + 2, buf0, g0)
      wait_s(buf1, s1)
      gather(j0 + 3, buf1, g1)
      return carry

    lax.fori_loop(0, (PH - 2) // 2, step, 0)
    wait_g(buf0, g0)
    scatter(base + PH - 2, buf0, s0)
    wait_g(buf1, g1)
    scatter(base + PH - 1, buf1, s1)

  wait_s(buf0, s0)
  wait_s(buf1, s1)
  plsc.subcore_barrier()
  _writeback(acc_sh, out_acc, c, s)


_agg = pl.kernel(
    _agg_body,
    out_type=jax.ShapeDtypeStruct((NC * N, D), jnp.float32),
    mesh=_MESH,
    scratch_types=(
        pltpu.VMEM((PH, CL), jnp.int32),
        pltpu.VMEM((CPL, CL), jnp.int32),
        pltpu.VMEM((CL, D), jnp.float32),
        pltpu.VMEM((CL, D), jnp.float32),
        pltpu.SemaphoreType.DMA,
        pltpu.SemaphoreType.DMA,
        pltpu.SemaphoreType.DMA,
        pltpu.SemaphoreType.DMA,
        pltpu.VMEM_SHARED((N, D), jnp.float32),
    ))


def _agg_cnt_body(y, src4, dst3, zeros_nw, ones_cd, out_acc, out_cnt,
                  idx_s, idx_d, buf0, buf1, g0, g1, s0, s1, acc_sh):
  c = lax.axis_index("c")
  s = lax.axis_index("s")
  w = s * NC + c
  _init_shared(zeros_nw, acc_sh, s)
  pltpu.sync_copy(dst3.at[w], idx_d)
  plsc.subcore_barrier()

  def gather(j, buf, sem):
    pltpu.async_copy(y.at[idx_s.at[j]], buf, sem)

  def scatter(g, buf, sem):
    pltpu.async_copy(buf, acc_sh.at[idx_d.at[g]], sem, add=True)

  def wait_g(buf, sem):
    pltpu.make_async_copy(y.at[idx_s.at[0]], buf, sem).wait()

  def wait_s(buf, sem):
    pltpu.make_async_copy(buf, acc_sh.at[idx_d.at[0]], sem).wait()

  for p in range(2):
    pltpu.sync_copy(src4.at[2 * w + p], idx_s)
    if p > 0:
      wait_s(buf0, s0)
    gather(0, buf0, g0)
    if p > 0:
      wait_s(buf1, s1)
    gather(1, buf1, g1)
    base = p * PH

    def step(i, carry):
      j0 = 2 * i
      wait_g(buf0, g0)
      scatter(base + j0, buf0, s0)
      wait_g(buf1, g1)
      scatter(base + j0 + 1, buf1, s1)
      wait_s(buf0, s0)
      gather(j0 + 2, buf0, g0)
      wait_s(buf1, s1)
      gather(j0 + 3, buf1, g1)
      return carry

    lax.fori_loop(0, (PH - 2) // 2, step, 0)
    wait_g(buf0, g0)
    scatter(base + PH - 2, buf0, s0)
    wait_g(buf1, g1)
    scatter(base + PH - 1, buf1, s1)

  wait_s(buf0, s0)
  wait_s(buf1, s1)
  plsc.subcore_barrier()
  _writeback(acc_sh, out_acc, c, s)
  plsc.subcore_barrier()

  _init_shared(zeros_nw, acc_sh, s)
  pltpu.sync_copy(ones_cd, buf0)
  plsc.subcore_barrier()

  def cscatter(g, sem):
    pltpu.async_copy(buf0, acc_sh.at[idx_d.at[g]], sem, add=True)

  def cwait(sem):
    pltpu.make_async_copy(buf0, acc_sh.at[idx_d.at[0]], sem).wait()

  csems = (g0, g1, s0, s1)
  for b in range(4):
    cscatter(b, csems[b])

  def cstep(i, carry):
    for b in range(4):
      cwait(csems[b])
      cscatter(4 * i + 4 + b, csems[b])
    return carry

  lax.fori_loop(0, (CPL - 4) // 4, cstep, 0)
  for b in range(4):
    cwait(csems[b])
  plsc.subcore_barrier()
  _writeback(acc_sh, out_cnt, c, s)


_agg_cnt = pl.kernel(
    _agg_cnt_body,
    out_type=(jax.ShapeDtypeStruct((NC * N, D), jnp.float32),
              jax.ShapeDtypeStruct((NC * N, D), jnp.float32)),
    mesh=_MESH,
    scratch_types=(
        pltpu.VMEM((PH, CL), jnp.int32),
        pltpu.VMEM((CPL, CL), jnp.int32),
        pltpu.VMEM((CL, D), jnp.float32),
        pltpu.VMEM((CL, D), jnp.float32),
        pltpu.SemaphoreType.DMA,
        pltpu.SemaphoreType.DMA,
        pltpu.SemaphoreType.DMA,
        pltpu.SemaphoreType.DMA,
        pltpu.VMEM_SHARED((N, D), jnp.float32),
    ))


def _dense1_body(a0_ref, a1_ref, c0_ref, c1_ref, x_ref, wl_ref, wr_ref,
                 bl_ref, g_ref, b_ref, o_ref, cnt_ref):
  acc = a0_ref[...] + a1_ref[...]
  cnt = jnp.maximum(c0_ref[:, 0:1] + c1_ref[:, 0:1], 1.0)
  mean = acc / cnt
  xb = x_ref[...]
  h = (jax.lax.dot(mean, wl_ref[...], precision=lax.Precision.HIGHEST,
                   preferred_element_type=jnp.float32)
       + bl_ref[...]
       + jax.lax.dot(xb, wr_ref[...], precision=lax.Precision.HIGHEST,
                     preferred_element_type=jnp.float32))
  mu = jnp.mean(h, axis=-1, keepdims=True)
  d = h - mu
  var = jnp.mean(d * d, axis=-1, keepdims=True)
  hn = d * jax.lax.rsqrt(var + 1e-5) * g_ref[...] + b_ref[...]
  o_ref[...] = jnp.maximum(hn, 0.0) + xb
  cnt_ref[...] = jnp.broadcast_to(cnt, cnt_ref.shape)


def _dense2_body(a0_ref, a1_ref, cnt_ref, x_ref, wl_ref, wr_ref, bl_ref,
                 g_ref, b_ref, o_ref):
  acc = a0_ref[...] + a1_ref[...]
  mean = acc / cnt_ref[:, 0:1]
  xb = x_ref[...]
  h = (jax.lax.dot(mean, wl_ref[...], precision=lax.Precision.HIGHEST,
                   preferred_element_type=jnp.float32)
       + bl_ref[...]
       + jax.lax.dot(xb, wr_ref[...], precision=lax.Precision.HIGHEST,
                     preferred_element_type=jnp.float32))
  mu = jnp.mean(h, axis=-1, keepdims=True)
  d = h - mu
  var = jnp.mean(d * d, axis=-1, keepdims=True)
  hn = d * jax.lax.rsqrt(var + 1e-5) * g_ref[...] + b_ref[...]
  o_ref[...] = hn + xb


R = 2000
NB = N // R

_W_SPEC = pl.BlockSpec((D, D), lambda i: (0, 0))
_B_SPEC = pl.BlockSpec((1, D), lambda i: (0, 0))
_ROW_SPEC = pl.BlockSpec((R, D), lambda i: (i, 0))
_LO_SPEC = pl.BlockSpec((R, D), lambda i: (i, 0))
_HI_SPEC = pl.BlockSpec((R, D), lambda i: (NB + i, 0))

_dense1 = pl.pallas_call(
    _dense1_body,
    grid=(NB,),
    in_specs=[
        _LO_SPEC, _HI_SPEC, _LO_SPEC, _HI_SPEC, _ROW_SPEC,
        _W_SPEC, _W_SPEC, _B_SPEC, _B_SPEC, _B_SPEC,
    ],
    out_specs=[
        _ROW_SPEC,
        pl.BlockSpec((R, 8), lambda i: (i, 0)),
    ],
    out_shape=[
        jax.ShapeDtypeStruct((N, D), jnp.float32),
        jax.ShapeDtypeStruct((N, 8), jnp.float32),
    ],
)

_dense2 = pl.pallas_call(
    _dense2_body,
    grid=(NB,),
    in_specs=[
        _LO_SPEC, _HI_SPEC,
        pl.BlockSpec((R, 8), lambda i: (i, 0)),
        _ROW_SPEC,
        _W_SPEC, _W_SPEC, _B_SPEC, _B_SPEC, _B_SPEC,
    ],
    out_specs=_ROW_SPEC,
    out_shape=jax.ShapeDtypeStruct((N, D), jnp.float32),
)


@jax.jit
def kernel(x, edge_index, Wl1, bl1, Wr1, g1, b1, Wl2, bl2, Wr2, g2, b2):
  src = edge_index[0]
  dst = edge_index[1]
  src4 = src.reshape(NW * 2, PH, CL)
  dst3 = dst.reshape(NW, CPL, CL)
  zeros_nd = jnp.zeros((N, D), jnp.float32)
  ones_cd = jnp.ones((CL, D), jnp.float32)

  acc1, cntp = _agg_cnt(x, src4, dst3, zeros_nd, ones_cd)
  h, cnt8 = _dense1(acc1, acc1, cntp, cntp, x, Wl1.T, Wr1.T,
                    bl1.reshape(1, D), g1.reshape(1, D), b1.reshape(1, D))
  acc2 = _agg(h, src4, dst3, zeros_nd)
  out = _dense2(acc2, acc2, cnt8, h, Wl2.T, Wr2.T, bl2.reshape(1, D),
                g2.reshape(1, D), b2.reshape(1, D))
  return out
